# baseline (speedup 1.0000x reference)
import functools

import jax
import jax.numpy as jnp
from jax import lax
from jax.experimental import pallas as pl
from jax.experimental.pallas import tpu as pltpu
from jax.experimental.pallas import tpu_sc as plsc

D = 128
CHUNK = 128
G = 2
NBUF = 3

_info = plsc.get_sparse_core_info()
NC, NS = _info.num_cores, _info.num_subcores
NW = NC * NS


@functools.partial(jax.jit, static_argnums=(2, 3))
def _sc_gather(table, idx_flat, batch, hist):
    n_chunks = hist // G
    per_w = CHUNK * hist
    mesh = plsc.VectorSubcoreMesh(core_axis_name="c", subcore_axis_name="s")

    @functools.partial(
        pl.kernel,
        mesh=mesh,
        out_type=jax.ShapeDtypeStruct((hist, batch, D), jnp.float32),
        scratch_types=[
            pltpu.VMEM((per_w,), jnp.int32),
        ]
        + [pltpu.VMEM((G * CHUNK, D), jnp.float32)] * NBUF
        + [pltpu.SemaphoreType.DMA] * (2 * NBUF),
    )
    def k(table_hbm, idx_hbm, out_hbm, idx_v, *bufs_sems):
        bufs = bufs_sems[:NBUF]
        gsem = bufs_sems[NBUF : 2 * NBUF]
        wsem = bufs_sems[2 * NBUF :]
        wid = lax.axis_index("s") * NC + lax.axis_index("c")
        base_b = wid * CHUNK
        pltpu.sync_copy(idx_hbm.at[pl.ds(wid * per_w, per_w)], idx_v)

        def g_start(p, b):
            pltpu.async_copy(
                table_hbm.at[idx_v.at[pl.ds(p * G * CHUNK, G * CHUNK)]],
                bufs[b],
                gsem[b],
            )

        def g_wait(b):
            pltpu.make_async_copy(
                table_hbm.at[idx_v.at[pl.ds(0, G * CHUNK)]], bufs[b], gsem[b]
            ).wait()

        def w_start(p, b):
            for gg in range(G):
                pltpu.async_copy(
                    bufs[b].at[pl.ds(gg * CHUNK, CHUNK)],
                    out_hbm.at[p * G + gg, pl.ds(base_b, CHUNK)],
                    wsem[b],
                )

        def w_wait(b):
            for gg in range(G):
                pltpu.make_async_copy(
                    bufs[b].at[pl.ds(gg * CHUNK, CHUNK)],
                    out_hbm.at[0, pl.ds(base_b, CHUNK)],
                    wsem[b],
                ).wait()

        for h in range(NBUF):
            g_start(h, h)
        g_wait(0)
        w_start(0, 0)

        def body(j, carry):
            h0 = NBUF * j + 1
            for t in range(NBUF):
                h = h0 + t
                b = (1 + t) % NBUF
                bp = t % NBUF
                g_wait(b)
                w_start(h, b)
                w_wait(bp)
                g_start(h + NBUF - 1, bp)
            return carry

        n_steady = (n_chunks - NBUF) // NBUF
        lax.fori_loop(0, n_steady, body, 0)

        for h in range(n_steady * NBUF + 1, n_chunks):
            b = h % NBUF
            g_wait(b)
            w_start(h, b)
            if h + NBUF - 1 < n_chunks:
                bp = (h - 1) % NBUF
                w_wait(bp)
                g_start(h + NBUF - 1, bp)
        for h in range(n_chunks - NBUF, n_chunks):
            w_wait(h % NBUF)

    return k(table, idx_flat)


def kernel(inputs, embeddings):
    batch, hist = inputs.shape
    idx = inputs.astype(jnp.int32)
    idx_arr = (
        idx.reshape(NW, CHUNK, hist).transpose(0, 2, 1).reshape(-1)
    )
    out = _sc_gather(embeddings, idx_arr, batch, hist)
    return out.transpose(1, 0, 2)


# speedup vs baseline: 1.0244x; 1.0244x over previous
import functools

import jax
import jax.numpy as jnp
from jax import lax
from jax.experimental import pallas as pl
from jax.experimental.pallas import tpu as pltpu
from jax.experimental.pallas import tpu_sc as plsc

D = 128
CHUNK = 128
G = 1
NBUF = 7

_info = plsc.get_sparse_core_info()
NC, NS = _info.num_cores, _info.num_subcores
NW = NC * NS


@functools.partial(jax.jit, static_argnums=(2, 3))
def _sc_gather(table, idx_flat, batch, hist):
    n_chunks = hist // G
    per_w = CHUNK * hist
    mesh = plsc.VectorSubcoreMesh(core_axis_name="c", subcore_axis_name="s")

    @functools.partial(
        pl.kernel,
        mesh=mesh,
        out_type=jax.ShapeDtypeStruct((hist, batch, D), jnp.float32),
        scratch_types=[
            pltpu.VMEM((per_w,), jnp.int32),
        ]
        + [pltpu.VMEM((G * CHUNK, D), jnp.float32)] * NBUF
        + [pltpu.SemaphoreType.DMA] * (2 * NBUF),
    )
    def k(table_hbm, idx_hbm, out_hbm, idx_v, *bufs_sems):
        bufs = bufs_sems[:NBUF]
        gsem = bufs_sems[NBUF : 2 * NBUF]
        wsem = bufs_sems[2 * NBUF :]
        wid = lax.axis_index("s") * NC + lax.axis_index("c")
        base_b = wid * CHUNK
        pltpu.sync_copy(idx_hbm.at[pl.ds(wid * per_w, per_w)], idx_v)

        def g_start(p, b):
            pltpu.async_copy(
                table_hbm.at[idx_v.at[pl.ds(p * G * CHUNK, G * CHUNK)]],
                bufs[b],
                gsem[b],
            )

        def g_wait(b):
            pltpu.make_async_copy(
                table_hbm.at[idx_v.at[pl.ds(0, G * CHUNK)]], bufs[b], gsem[b]
            ).wait()

        def w_start(p, b):
            for gg in range(G):
                pltpu.async_copy(
                    bufs[b].at[pl.ds(gg * CHUNK, CHUNK)],
                    out_hbm.at[p * G + gg, pl.ds(base_b, CHUNK)],
                    wsem[b],
                )

        def w_wait(b):
            for gg in range(G):
                pltpu.make_async_copy(
                    bufs[b].at[pl.ds(gg * CHUNK, CHUNK)],
                    out_hbm.at[0, pl.ds(base_b, CHUNK)],
                    wsem[b],
                ).wait()

        for h in range(NBUF):
            g_start(h, h)
        g_wait(0)
        w_start(0, 0)

        def body(j, carry):
            h0 = NBUF * j + 1
            for t in range(NBUF):
                h = h0 + t
                b = (1 + t) % NBUF
                bp = t % NBUF
                g_wait(b)
                w_start(h, b)
                w_wait(bp)
                g_start(h + NBUF - 1, bp)
            return carry

        n_steady = (n_chunks - NBUF) // NBUF
        lax.fori_loop(0, n_steady, body, 0)

        for h in range(n_steady * NBUF + 1, n_chunks):
            b = h % NBUF
            g_wait(b)
            w_start(h, b)
            if h + NBUF - 1 < n_chunks:
                bp = (h - 1) % NBUF
                w_wait(bp)
                g_start(h + NBUF - 1, bp)
        for h in range(n_chunks - NBUF, n_chunks):
            w_wait(h % NBUF)

    return k(table, idx_flat)


def kernel(inputs, embeddings):
    batch, hist = inputs.shape
    idx = inputs.astype(jnp.int32)
    idx_arr = (
        idx.reshape(NW, CHUNK, hist).transpose(0, 2, 1).reshape(-1)
    )
    out = _sc_gather(embeddings, idx_arr, batch, hist)
    return out.transpose(1, 0, 2)
